# Initial kernel scaffold; baseline (speedup 1.0000x reference)
import jax
import jax.numpy as jnp
from jax.experimental import pallas as pl


def kernel(batch_box_preds, batch_cls_preds, batch_size):
    raise NotImplementedError("write your pallas kernel here")


# speedup vs baseline: 3.5781x; 3.5781x over previous
import jax
import jax.numpy as jnp
from jax.experimental import pallas as pl
from jax.experimental.pallas import tpu as pltpu

_NMS_PRE = 2048
_NMS_POST = 512
_NMS_THRESH = 0.7
_NUM_CLASS = 3


def _nms_body(bt_ref, pk_ref,
              rois_ref, scr_ref, labo_ref, lgo_ref,
              ab, kps):
    x = bt_ref[0, 0:1, :]
    y = bt_ref[0, 1:2, :]
    dx = bt_ref[0, 3:4, :]
    dy = bt_ref[0, 4:5, :]
    ry = bt_ref[0, 6:7, :]
    c = jnp.abs(jnp.cos(ry))
    s = jnp.abs(jnp.sin(ry))
    hw = 0.5 * (dx * c + dy * s)
    hh = 0.5 * (dx * s + dy * c)
    x1 = x - hw
    y1 = y - hh
    x2 = x + hw
    y2 = y + hh
    area = (x2 - x1) * (y2 - y1)
    ab[...] = jnp.concatenate(
        [x1, y1, x2, y2, area, jnp.zeros((3, _NMS_PRE), jnp.float32)], axis=0)
    kps[...] = jnp.ones_like(kps)

    idxv = jax.lax.broadcasted_iota(jnp.int32, (1, _NMS_PRE), 1)

    def body(i, carry):
        count, dstv = carry
        onehot = (idxv == i).astype(jnp.float32)
        cols = jnp.sum(ab[...] * onehot, axis=1, keepdims=True)
        keep_i = jnp.sum(kps[...] * onehot, axis=1, keepdims=True)
        x1i = cols[0:1, :]
        y1i = cols[1:2, :]
        x2i = cols[2:3, :]
        y2i = cols[3:4, :]
        ai = cols[4:5, :]
        abv = ab[...]
        ix1 = jnp.maximum(abv[0:1, :], x1i)
        iy1 = jnp.maximum(abv[1:2, :], y1i)
        ix2 = jnp.minimum(abv[2:3, :], x2i)
        iy2 = jnp.minimum(abv[3:4, :], y2i)
        inter = jnp.maximum(ix2 - ix1, 0.0) * jnp.maximum(iy2 - iy1, 0.0)
        iou = inter / (ai + abv[4:5, :] - inter + 1e-8)
        alive = keep_i > 0.0
        sup = (iou > _NMS_THRESH) & (idxv > i) & alive
        kps[...] = jnp.where(sup, 0.0, kps[...])
        do_store = alive & (count < _NMS_POST)
        dstv = jnp.where((idxv == i) & do_store, count, dstv)
        return count + do_store.astype(jnp.int32), dstv

    _, dstv = jax.lax.fori_loop(
        0, _NMS_PRE, body,
        (jnp.zeros((1, 1), jnp.int32),
         jnp.full((1, _NMS_PRE), _NMS_POST, jnp.int32)))

    slot = jax.lax.broadcasted_iota(jnp.int32, (_NMS_POST, _NMS_PRE), 0)
    sel = (slot == jnp.broadcast_to(dstv, (_NMS_POST, _NMS_PRE))
           ).astype(jnp.float32)
    out = jnp.dot(sel, pk_ref[0], preferred_element_type=jnp.float32)
    rois_ref[0] = out[:, 0:7]
    scr_ref[0] = out[:, 7:8]
    labo_ref[0] = out[:, 11:12].astype(jnp.int32)
    lgo_ref[0] = out[:, 8:11]


@jax.jit
def _run(bt, pk):
    B = bt.shape[0]
    return pl.pallas_call(
        _nms_body,
        grid=(B,),
        in_specs=[
            pl.BlockSpec((1, 8, _NMS_PRE), lambda b: (b, 0, 0)),
            pl.BlockSpec((1, _NMS_PRE, 16), lambda b: (b, 0, 0)),
        ],
        out_specs=[
            pl.BlockSpec((1, _NMS_POST, 7), lambda b: (b, 0, 0)),
            pl.BlockSpec((1, _NMS_POST, 1), lambda b: (b, 0, 0)),
            pl.BlockSpec((1, _NMS_POST, 1), lambda b: (b, 0, 0)),
            pl.BlockSpec((1, _NMS_POST, _NUM_CLASS), lambda b: (b, 0, 0)),
        ],
        out_shape=[
            jax.ShapeDtypeStruct((B, _NMS_POST, 7), jnp.float32),
            jax.ShapeDtypeStruct((B, _NMS_POST, 1), jnp.float32),
            jax.ShapeDtypeStruct((B, _NMS_POST, 1), jnp.int32),
            jax.ShapeDtypeStruct((B, _NMS_POST, _NUM_CLASS), jnp.float32),
        ],
        scratch_shapes=[pltpu.VMEM((8, _NMS_PRE), jnp.float32),
                        pltpu.VMEM((1, _NMS_PRE), jnp.float32)],
    )(bt, pk)


def kernel(batch_box_preds, batch_cls_preds, batch_size):
    scores = jnp.max(batch_cls_preds, axis=-1)
    labels = jnp.argmax(batch_cls_preds, axis=-1).astype(jnp.int32)
    top_s, order = jax.lax.top_k(scores, _NMS_PRE)
    b = jnp.take_along_axis(batch_box_preds, order[..., None], axis=1)
    lg = jnp.take_along_axis(batch_cls_preds, order[..., None], axis=1)
    lab = jnp.take_along_axis(labels, order, axis=1)
    B = b.shape[0]
    bt = jnp.concatenate(
        [jnp.transpose(b, (0, 2, 1)), top_s[:, None, :]], axis=1)
    pk = jnp.concatenate(
        [b, top_s[..., None], lg, (lab + 1).astype(jnp.float32)[..., None],
         jnp.zeros((B, _NMS_PRE, 4), jnp.float32)], axis=-1)
    rois, scr, labo, lgo = _run(bt, pk)
    return rois, scr[:, :, 0], labo[:, :, 0], lgo


# speedup vs baseline: 9.5697x; 2.6745x over previous
import jax
import jax.numpy as jnp
from jax.experimental import pallas as pl
from jax.experimental.pallas import tpu as pltpu

_NMS_PRE = 2048
_NMS_POST = 512
_NMS_THRESH = 0.7
_NUM_CLASS = 3
_SUB = 16
_LANE = 128


def _nms_body(bt_ref, pk_ref,
              rois_ref, scr_ref, labo_ref, lgo_ref,
              x1s, y1s, x2s, y2s, ars, kps):
    x = bt_ref[0, 0]
    y = bt_ref[0, 1]
    dx = bt_ref[0, 3]
    dy = bt_ref[0, 4]
    ry = bt_ref[0, 6]
    c = jnp.abs(jnp.cos(ry))
    s = jnp.abs(jnp.sin(ry))
    hw = 0.5 * (dx * c + dy * s)
    hh = 0.5 * (dx * s + dy * c)
    x1 = x - hw
    y1 = y - hh
    x2 = x + hw
    y2 = y + hh
    x1s[...] = x1
    y1s[...] = y1
    x2s[...] = x2
    y2s[...] = y2
    ars[...] = (x2 - x1) * (y2 - y1)
    kps[...] = jnp.ones_like(kps)

    idx = (jax.lax.broadcasted_iota(jnp.int32, (_SUB, _LANE), 0) * _LANE
           + jax.lax.broadcasted_iota(jnp.int32, (_SUB, _LANE), 1))

    def cond(carry):
        i, count, _ = carry
        return (i < _NMS_PRE) & (count < _NMS_POST)

    def body(carry):
        i, count, dstv = carry
        oneb = idx == i
        onehot = oneb.astype(jnp.float32)
        keep_i = jnp.sum(kps[...] * onehot)
        x1i = jnp.sum(x1s[...] * onehot)
        y1i = jnp.sum(y1s[...] * onehot)
        x2i = jnp.sum(x2s[...] * onehot)
        y2i = jnp.sum(y2s[...] * onehot)
        ai = jnp.sum(ars[...] * onehot)
        ix1 = jnp.maximum(x1s[...], x1i)
        iy1 = jnp.maximum(y1s[...], y1i)
        ix2 = jnp.minimum(x2s[...], x2i)
        iy2 = jnp.minimum(y2s[...], y2i)
        inter = jnp.maximum(ix2 - ix1, 0.0) * jnp.maximum(iy2 - iy1, 0.0)
        iou = inter / (ai + ars[...] - inter + 1e-8)
        alive = keep_i > 0.0
        sup = (iou > _NMS_THRESH) & (idx > i) & alive
        kps[...] = jnp.where(sup, 0.0, kps[...])
        dstv = jnp.where(oneb & alive, count, dstv)
        return i + 1, count + alive.astype(jnp.int32), dstv

    _, _, dstv = jax.lax.while_loop(
        cond, body,
        (jnp.int32(0), jnp.int32(0),
         jnp.full((_SUB, _LANE), _NMS_POST, jnp.int32)))

    slot = jax.lax.broadcasted_iota(jnp.int32, (_NMS_POST, _LANE), 0)
    out = jnp.zeros((_NMS_POST, 16), jnp.float32)
    for k in range(_SUB):
        sel = (slot == jnp.broadcast_to(
            dstv[k:k + 1, :], (_NMS_POST, _LANE))).astype(jnp.float32)
        out = out + jnp.dot(sel, pk_ref[0, k * _LANE:(k + 1) * _LANE, :],
                            preferred_element_type=jnp.float32)
    rois_ref[0] = out[:, 0:7]
    scr_ref[0] = out[:, 7:8]
    labo_ref[0] = out[:, 11:12].astype(jnp.int32)
    lgo_ref[0] = out[:, 8:11]


@jax.jit
def _run(bt, pk):
    B = bt.shape[0]
    return pl.pallas_call(
        _nms_body,
        grid=(B,),
        in_specs=[
            pl.BlockSpec((1, 8, _SUB, _LANE), lambda b: (b, 0, 0, 0)),
            pl.BlockSpec((1, _NMS_PRE, 16), lambda b: (b, 0, 0)),
        ],
        out_specs=[
            pl.BlockSpec((1, _NMS_POST, 7), lambda b: (b, 0, 0)),
            pl.BlockSpec((1, _NMS_POST, 1), lambda b: (b, 0, 0)),
            pl.BlockSpec((1, _NMS_POST, 1), lambda b: (b, 0, 0)),
            pl.BlockSpec((1, _NMS_POST, _NUM_CLASS), lambda b: (b, 0, 0)),
        ],
        out_shape=[
            jax.ShapeDtypeStruct((B, _NMS_POST, 7), jnp.float32),
            jax.ShapeDtypeStruct((B, _NMS_POST, 1), jnp.float32),
            jax.ShapeDtypeStruct((B, _NMS_POST, 1), jnp.int32),
            jax.ShapeDtypeStruct((B, _NMS_POST, _NUM_CLASS), jnp.float32),
        ],
        scratch_shapes=[pltpu.VMEM((_SUB, _LANE), jnp.float32)
                        for _ in range(6)],
        compiler_params=pltpu.CompilerParams(
            dimension_semantics=("parallel",)),
    )(bt, pk)


def kernel(batch_box_preds, batch_cls_preds, batch_size):
    scores = jnp.max(batch_cls_preds, axis=-1)
    labels = jnp.argmax(batch_cls_preds, axis=-1).astype(jnp.int32)
    top_s, order = jax.lax.top_k(scores, _NMS_PRE)
    b = jnp.take_along_axis(batch_box_preds, order[..., None], axis=1)
    lg = jnp.take_along_axis(batch_cls_preds, order[..., None], axis=1)
    lab = jnp.take_along_axis(labels, order, axis=1)
    B = b.shape[0]
    bt = jnp.concatenate(
        [jnp.transpose(b, (0, 2, 1)), top_s[:, None, :]],
        axis=1).reshape(B, 8, _SUB, _LANE)
    pk = jnp.concatenate(
        [b, top_s[..., None], lg, (lab + 1).astype(jnp.float32)[..., None],
         jnp.zeros((B, _NMS_PRE, 4), jnp.float32)], axis=-1)
    rois, scr, labo, lgo = _run(bt, pk)
    return rois, scr[:, :, 0], labo[:, :, 0], lgo


# speedup vs baseline: 10.2648x; 1.0726x over previous
import jax
import jax.numpy as jnp
from jax.experimental import pallas as pl
from jax.experimental.pallas import tpu as pltpu

_NMS_PRE = 2048
_NMS_POST = 512
_NMS_THRESH = 0.7
_NUM_CLASS = 3
_SUB = 16
_LANE = 128


def _nms_body(abv_ref, absm_ref, pk_ref,
              rois_ref, scr_ref, labo_ref, lgo_ref,
              kps):
    kps[...] = jnp.ones_like(kps)
    x1v = abv_ref[0, 0]
    y1v = abv_ref[0, 1]
    x2v = abv_ref[0, 2]
    y2v = abv_ref[0, 3]
    arv = abv_ref[0, 4]

    idx = (jax.lax.broadcasted_iota(jnp.int32, (_SUB, _LANE), 0) * _LANE
           + jax.lax.broadcasted_iota(jnp.int32, (_SUB, _LANE), 1))

    def cond(carry):
        i, count, _ = carry
        return (i < _NMS_PRE) & (count < _NMS_POST)

    def body(carry):
        i, count, dstv = carry
        oneb = idx == i
        keep_i = jnp.sum(kps[...] * oneb.astype(jnp.float32))
        x1i = absm_ref[0, 0, i]
        y1i = absm_ref[0, 1, i]
        x2i = absm_ref[0, 2, i]
        y2i = absm_ref[0, 3, i]
        ai = absm_ref[0, 4, i]
        ix1 = jnp.maximum(x1v, x1i)
        iy1 = jnp.maximum(y1v, y1i)
        ix2 = jnp.minimum(x2v, x2i)
        iy2 = jnp.minimum(y2v, y2i)
        inter = jnp.maximum(ix2 - ix1, 0.0) * jnp.maximum(iy2 - iy1, 0.0)
        iou = inter / (ai + arv - inter + 1e-8)
        alive = keep_i > 0.0
        sup = (iou > _NMS_THRESH) & alive
        kps[...] = jnp.where(sup, 0.0, kps[...])
        dstv = jnp.where(oneb & alive, count, dstv)
        return i + 1, count + alive.astype(jnp.int32), dstv

    _, _, dstv = jax.lax.while_loop(
        cond, body,
        (jnp.int32(0), jnp.int32(0),
         jnp.full((_SUB, _LANE), _NMS_POST, jnp.int32)))

    slot = jax.lax.broadcasted_iota(jnp.int32, (_NMS_POST, _LANE), 0)
    out = jnp.zeros((_NMS_POST, 16), jnp.float32)
    for k in range(_SUB):
        sel = (slot == jnp.broadcast_to(
            dstv[k:k + 1, :], (_NMS_POST, _LANE))).astype(jnp.float32)
        out = out + jnp.dot(sel, pk_ref[0, k * _LANE:(k + 1) * _LANE, :],
                            preferred_element_type=jnp.float32)
    rois_ref[0] = out[:, 0:7]
    scr_ref[0] = out[:, 7:8]
    labo_ref[0] = out[:, 11:12].astype(jnp.int32)
    lgo_ref[0] = out[:, 8:11]


@jax.jit
def _run(abv, absm, pk):
    B = abv.shape[0]
    return pl.pallas_call(
        _nms_body,
        grid=(B,),
        in_specs=[
            pl.BlockSpec((1, 5, _SUB, _LANE), lambda b: (b, 0, 0, 0)),
            pl.BlockSpec((1, 5, _NMS_PRE), lambda b: (b, 0, 0),
                         memory_space=pltpu.SMEM),
            pl.BlockSpec((1, _NMS_PRE, 16), lambda b: (b, 0, 0)),
        ],
        out_specs=[
            pl.BlockSpec((1, _NMS_POST, 7), lambda b: (b, 0, 0)),
            pl.BlockSpec((1, _NMS_POST, 1), lambda b: (b, 0, 0)),
            pl.BlockSpec((1, _NMS_POST, 1), lambda b: (b, 0, 0)),
            pl.BlockSpec((1, _NMS_POST, _NUM_CLASS), lambda b: (b, 0, 0)),
        ],
        out_shape=[
            jax.ShapeDtypeStruct((B, _NMS_POST, 7), jnp.float32),
            jax.ShapeDtypeStruct((B, _NMS_POST, 1), jnp.float32),
            jax.ShapeDtypeStruct((B, _NMS_POST, 1), jnp.int32),
            jax.ShapeDtypeStruct((B, _NMS_POST, _NUM_CLASS), jnp.float32),
        ],
        scratch_shapes=[pltpu.VMEM((_SUB, _LANE), jnp.float32)],
        compiler_params=pltpu.CompilerParams(
            dimension_semantics=("parallel",)),
    )(abv, absm, pk)


def kernel(batch_box_preds, batch_cls_preds, batch_size):
    scores = jnp.max(batch_cls_preds, axis=-1)
    _, order = jax.lax.top_k(scores, _NMS_PRE)
    payload = jnp.concatenate([batch_box_preds, batch_cls_preds], axis=-1)
    g = jnp.take_along_axis(payload, order[..., None], axis=1)
    b = g[..., 0:7]
    lg = g[..., 7:10]
    top_s = jnp.max(lg, axis=-1)
    lab = jnp.argmax(lg, axis=-1).astype(jnp.int32)
    B = b.shape[0]
    x = b[..., 0]
    y = b[..., 1]
    dx = b[..., 3]
    dy = b[..., 4]
    ry = b[..., 6]
    c = jnp.abs(jnp.cos(ry))
    s = jnp.abs(jnp.sin(ry))
    hw = 0.5 * (dx * c + dy * s)
    hh = 0.5 * (dx * s + dy * c)
    x1 = x - hw
    y1 = y - hh
    x2 = x + hw
    y2 = y + hh
    ab5 = jnp.stack([x1, y1, x2, y2, (x2 - x1) * (y2 - y1)], axis=1)
    abv = ab5.reshape(B, 5, _SUB, _LANE)
    pk = jnp.concatenate(
        [b, top_s[..., None], lg, (lab + 1).astype(jnp.float32)[..., None],
         jnp.zeros((B, _NMS_PRE, 4), jnp.float32)], axis=-1)
    rois, scr, labo, lgo = _run(abv, ab5, pk)
    return rois, scr[:, :, 0], labo[:, :, 0], lgo
